# baseline (speedup 1.0000x reference)
import jax
import jax.numpy as jnp
from jax.experimental import pallas as pl
from jax.experimental.pallas import tpu as pltpu
from jax.experimental.pallas import tpu_sc as plsc

_NUM_STUDENTS = 4096
_NUM_QUESTIONS = 16384
_NUM_CONCEPTS = 128
_QB = 1024
_H = _NUM_QUESTIONS // 2

_NC, _NS, _L = 2, 16, 16
_NW = _NC * _NS
_BPW = _H // _NW


def _dgather(dr_hbm, c_hbm, d_hbm, c_v, ridx_v, out_v, sem):
    wid = jax.lax.axis_index("s") * _NC + jax.lax.axis_index("c")
    base = wid * _BPW
    pltpu.sync_copy(c_hbm.at[pl.ds(base, _BPW)], c_v)
    iota = jax.lax.iota(jnp.int32, _L)
    for i in range(_BPW // _L):
        cv = c_v[pl.ds(i * _L, _L)]
        qv = iota + (base + i * _L)
        ridx_v[pl.ds(i * _L, _L)] = qv * _NUM_CONCEPTS + cv
    pltpu.async_copy(dr_hbm.at[ridx_v], out_v, sem).wait()
    pltpu.sync_copy(out_v, d_hbm.at[pl.ds(base, _BPW)])


def _fwd(a_ref, dg_ref, c_ref, o_ref):
    c = c_ref[0]
    oh = (c == jax.lax.broadcasted_iota(jnp.int32, (_NUM_CONCEPTS, _QB), 0))
    oh = oh.astype(jnp.bfloat16)
    neg_dh = (-0.5 * dg_ref[0]).astype(jnp.bfloat16)
    pad = jnp.zeros((_NUM_CONCEPTS - 1, _QB), jnp.bfloat16)
    oh_aug = jnp.concatenate([oh, neg_dh, pad], axis=0)
    z = jnp.dot(a_ref[...], oh_aug, preferred_element_type=jnp.float32)
    o_ref[...] = 0.6 + 0.4 * jnp.tanh(z)


def _fwd_aliased(a_ref, dg_ref, c_ref, prev_ref, o_ref):
    _fwd(a_ref, dg_ref, c_ref, o_ref)


def kernel(x, A, D, concepts_q, concepts_c):
    nb = _H // _QB
    dg = pl.kernel(
        _dgather,
        out_type=jax.ShapeDtypeStruct((_H,), jnp.float32),
        mesh=plsc.VectorSubcoreMesh(core_axis_name="c", subcore_axis_name="s"),
        scratch_types=[
            pltpu.VMEM((_BPW,), jnp.int32),
            pltpu.VMEM((_BPW,), jnp.int32),
            pltpu.VMEM((_BPW,), jnp.float32),
            pltpu.SemaphoreType.DMA,
        ],
    )
    d_flat = D.reshape(_NUM_QUESTIONS * _NUM_CONCEPTS)
    c1, c2 = concepts_c[:_H], concepts_c[_H:]
    d1 = dg(d_flat, c1)
    d2 = dg(d_flat, c2)

    a_aug = jnp.zeros((_NUM_STUDENTS, 2 * _NUM_CONCEPTS), jnp.bfloat16)
    a_aug = a_aug.at[:, :_NUM_CONCEPTS].set((0.5 * A).astype(jnp.bfloat16))
    a_aug = a_aug.at[:, _NUM_CONCEPTS].set(jnp.bfloat16(1.0))
    out_shape = jax.ShapeDtypeStruct((_NUM_STUDENTS, _NUM_QUESTIONS),
                                     jnp.float32)
    common_specs = [
        pl.BlockSpec((_NUM_STUDENTS, 2 * _NUM_CONCEPTS), lambda q: (0, 0)),
        pl.BlockSpec((1, 1, _QB), lambda q: (q, 0, 0)),
        pl.BlockSpec((1, 1, _QB), lambda q: (q, 0, 0)),
    ]
    half1 = pl.pallas_call(
        _fwd,
        grid=(nb,),
        in_specs=common_specs,
        out_specs=pl.BlockSpec((_NUM_STUDENTS, _QB), lambda q: (0, q)),
        out_shape=out_shape,
    )(a_aug, d1.reshape(nb, 1, _QB), c1.reshape(nb, 1, _QB))
    return pl.pallas_call(
        _fwd_aliased,
        grid=(nb,),
        in_specs=common_specs + [pl.BlockSpec(memory_space=pl.ANY)],
        out_specs=pl.BlockSpec((_NUM_STUDENTS, _QB), lambda q: (0, q + nb)),
        out_shape=out_shape,
        input_output_aliases={3: 0},
    )(a_aug, d2.reshape(nb, 1, _QB), c2.reshape(nb, 1, _QB), half1)


# speedup vs baseline: 1.0794x; 1.0794x over previous
import jax
import jax.numpy as jnp
from jax.experimental import pallas as pl
from jax.experimental.pallas import tpu as pltpu
from jax.experimental.pallas import tpu_sc as plsc

_NUM_STUDENTS = 4096
_NUM_QUESTIONS = 16384
_NUM_CONCEPTS = 128
_QB = 1024

_NC, _NS, _L = 2, 16, 16
_NW = _NC * _NS
_BPW = _NUM_QUESTIONS // _NW


def _dgather(dr_hbm, c_hbm, d_hbm, c_v, ridx_v, out_v, sem):
    wid = jax.lax.axis_index("s") * _NC + jax.lax.axis_index("c")
    base = wid * _BPW
    pltpu.sync_copy(c_hbm.at[pl.ds(base, _BPW)], c_v)
    iota = jax.lax.iota(jnp.int32, _L)
    for i in range(_BPW // _L):
        cv = c_v[pl.ds(i * _L, _L)]
        qv = iota + (base + i * _L)
        ridx_v[pl.ds(i * _L, _L)] = qv * _NUM_CONCEPTS + cv
    pltpu.async_copy(dr_hbm.at[ridx_v], out_v, sem).wait()
    pltpu.sync_copy(out_v, d_hbm.at[pl.ds(base, _BPW)])


def _fwd(a_ref, dg_ref, c_ref, o_ref):
    c = c_ref[0]
    oh = (c == jax.lax.broadcasted_iota(jnp.int32, (_NUM_CONCEPTS, _QB), 0))
    oh = oh.astype(jnp.bfloat16)
    neg_dh = (-0.5 * dg_ref[0]).astype(jnp.bfloat16)
    pad = jnp.zeros((_NUM_CONCEPTS - 1, _QB), jnp.bfloat16)
    oh_aug = jnp.concatenate([oh, neg_dh, pad], axis=0)
    z = jnp.dot(a_ref[...], oh_aug, preferred_element_type=jnp.float32)
    o_ref[...] = 0.6 + 0.4 * jnp.tanh(z)


def kernel(x, A, D, concepts_q, concepts_c):
    dg = pl.kernel(
        _dgather,
        out_type=jax.ShapeDtypeStruct((_NUM_QUESTIONS,), jnp.float32),
        mesh=plsc.VectorSubcoreMesh(core_axis_name="c", subcore_axis_name="s"),
        scratch_types=[
            pltpu.VMEM((_BPW,), jnp.int32),
            pltpu.VMEM((_BPW,), jnp.int32),
            pltpu.VMEM((_BPW,), jnp.float32),
            pltpu.SemaphoreType.DMA,
        ],
    )
    d = dg(D.reshape(_NUM_QUESTIONS * _NUM_CONCEPTS), concepts_c)

    nb = _NUM_QUESTIONS // _QB
    c3 = concepts_c.reshape(nb, 1, _QB)
    d3 = d.reshape(nb, 1, _QB)
    a_aug = jnp.zeros((_NUM_STUDENTS, 2 * _NUM_CONCEPTS), jnp.bfloat16)
    a_aug = a_aug.at[:, :_NUM_CONCEPTS].set((0.5 * A).astype(jnp.bfloat16))
    a_aug = a_aug.at[:, _NUM_CONCEPTS].set(jnp.bfloat16(1.0))
    return pl.pallas_call(
        _fwd,
        grid=(nb,),
        in_specs=[
            pl.BlockSpec((_NUM_STUDENTS, 2 * _NUM_CONCEPTS), lambda q: (0, 0)),
            pl.BlockSpec((1, 1, _QB), lambda q: (q, 0, 0)),
            pl.BlockSpec((1, 1, _QB), lambda q: (q, 0, 0)),
        ],
        out_specs=pl.BlockSpec((_NUM_STUDENTS, _QB), lambda q: (0, q)),
        out_shape=jax.ShapeDtypeStruct((_NUM_STUDENTS, _NUM_QUESTIONS),
                                       jnp.float32),
    )(a_aug, d3, c3)
